# Initial kernel scaffold; baseline (speedup 1.0000x reference)
import jax
import jax.numpy as jnp
from jax.experimental import pallas as pl


def kernel(input_ids, emb_table, q_amps, ent_mat, evo_W, evo_b, dec_W, dec_b, ln_g, ln_b):
    raise NotImplementedError("write your pallas kernel here")


# speedup vs baseline: 1.4452x; 1.4452x over previous
import functools

import numpy as np
import jax
import jax.numpy as jnp
from jax import lax
from jax.experimental import pallas as pl
from jax.experimental.pallas import tpu as pltpu
from jax.experimental.pallas import tpu_sc as plsc

VOCAB = 1000
DIM = 128
DEPTH = 3
MAXPOS = 512
DECO = 0.1
S = 50
SP = 64
VP = 1024
NC, NS, LANES = 2, 16, 16
NW = NC * NS
GB = 8


def _pos_encoding_np(max_len, d):
    position = np.arange(max_len, dtype=np.float32)[:, None]
    div_term = np.exp(np.arange(0, d, 2, dtype=np.float32) * -(np.log(10000.0) / d))
    pe = np.zeros((max_len, d), dtype=np.float32)
    pe[:, 0::2] = np.sin(position * div_term)
    pe[:, 1::2] = np.cos(position * div_term)
    quantum_phase = np.sin(position * np.pi / max_len)
    pe = pe * (1.0 + 0.1 * quantum_phase)
    return pe


def _np_consts(s):
    pe = np.zeros((SP, DIM), dtype=np.float32)
    pe[:s] = _pos_encoding_np(MAXPOS, DIM)[:s]
    pos = np.arange(s, dtype=np.float32)
    dist = np.abs(pos[None, :] - pos[:, None])
    decay = np.exp(-DECO * dist) * (1.0 - np.eye(s, dtype=np.float32))
    cmask = np.zeros((SP, SP), dtype=np.float32)
    cmask[:s, :s] = decay
    return jnp.asarray(pe), jnp.asarray(cmask)


def _sc_body(bpw, ids_hbm, tab_hbm, ent_hbm, g_hbm, c_hbm,
             idv, rows, erows, cbuf, sem_t, sem_e):
    wid = lax.axis_index("s") * NC + lax.axis_index("c")

    def per_batch(k, carry):
        b = wid * bpw + k
        pltpu.sync_copy(ids_hbm.at[b], idv)
        cp_t = pltpu.async_copy(tab_hbm.at[idv], rows, sem_t)
        cp_e = pltpu.async_copy(ent_hbm.at[idv], erows, sem_e)
        cp_e.wait()
        col_idx = [idv[pl.ds(g * LANES, LANES)] for g in range(SP // LANES)]

        def per_row(i, c2):
            row_splat = jnp.full((LANES,), i, dtype=jnp.int32)
            for g in range(SP // LANES):
                vals = plsc.load_gather(erows, [row_splat, col_idx[g]])
                cbuf[pl.ds(i * SP + g * LANES, LANES)] = vals
            return c2

        lax.fori_loop(0, SP, per_row, 0)
        cp_t.wait()
        pltpu.sync_copy(rows, g_hbm.at[b])
        pltpu.sync_copy(cbuf, c_hbm.at[b])
        return carry

    lax.fori_loop(0, bpw, per_batch, 0)


def _sc_gather(ids_pad, tab, ent_pad, batch):
    bpw = batch // NW
    mesh = plsc.VectorSubcoreMesh(core_axis_name="c", subcore_axis_name="s")
    fn = pl.kernel(
        functools.partial(_sc_body, bpw),
        out_type=[
            jax.ShapeDtypeStruct((batch, SP, 3 * DIM), jnp.float32),
            jax.ShapeDtypeStruct((batch, SP * SP), jnp.float32),
        ],
        mesh=mesh,
        compiler_params=pltpu.CompilerParams(needs_layout_passes=False),
        scratch_types=[
            pltpu.VMEM((SP,), jnp.int32),
            pltpu.VMEM((SP, 3 * DIM), jnp.float32),
            pltpu.VMEM((SP, VP), jnp.float32),
            pltpu.VMEM((SP * SP,), jnp.float32),
            pltpu.SemaphoreType.DMA,
            pltpu.SemaphoreType.DMA,
        ],
    )
    return fn(ids_pad, tab, ent_pad)


def _tc_body(s, g_ref, c_ref, pe_ref, cmask_ref, evo_w_ref, evo_b_ref,
             dec_w_ref, dec_b_ref, ln_g_ref, ln_b_ref, out_ref):
    g = g_ref[...]
    emb = g[:, :, :DIM]
    qr = g[:, :, DIM:2 * DIM]
    qi = g[:, :, 2 * DIM:]
    pe = pe_ref[...]
    rowm = (lax.broadcasted_iota(jnp.int32, (SP, 1), 0) < s).astype(jnp.float32)

    x0 = emb + pe[None]
    real = x0 + qr
    mag = jnp.sqrt(real * real + qi * qi + 1e-8)
    x = x0 + 0.1 * mag * jnp.tanh(real)
    x = x * rowm[None]

    cm = c_ref[...] * cmask_ref[...][None]
    dn = (((1,), (0,)), ((), ()))
    ys = [lax.dot_general(cm[k], x[k], dn, preferred_element_type=jnp.float32)
          for k in range(GB)]
    x = x + 0.1 * jnp.stack(ys)

    xf = x.reshape(GB * SP, DIM)
    dnt = (((1,), (1,)), ((), ()))
    for step in range(DEPTH):
        ev = (lax.dot_general(xf, evo_w_ref[step], dnt,
                              preferred_element_type=jnp.float32)
              + evo_b_ref[step][None])
        gate = jax.nn.sigmoid(
            lax.dot_general(xf, dec_w_ref[step], dnt,
                            preferred_element_type=jnp.float32)
            + dec_b_ref[step][None])
        xf = ev * gate + xf * (1.0 - gate)

    mu = jnp.mean(xf, axis=-1, keepdims=True)
    var = jnp.mean((xf - mu) ** 2, axis=-1, keepdims=True)
    out = (xf - mu) * lax.rsqrt(var + 1e-5) * ln_g_ref[...][None] + ln_b_ref[...][None]
    out_ref[...] = out.reshape(GB, SP, DIM)


def _tc_dense(s, batch, g, c, pe, cmask, evo_w, evo_b, dec_w, dec_b, ln_g, ln_b):
    grid = (batch // GB,)
    rep2 = lambda i: (0, 0)
    rep1 = lambda i: (0,)
    return pl.pallas_call(
        functools.partial(_tc_body, s),
        grid=grid,
        in_specs=[
            pl.BlockSpec((GB, SP, 3 * DIM), lambda i: (i, 0, 0)),
            pl.BlockSpec((GB, SP, SP), lambda i: (i, 0, 0)),
            pl.BlockSpec((SP, DIM), rep2),
            pl.BlockSpec((SP, SP), rep2),
            pl.BlockSpec((DEPTH, DIM, DIM), lambda i: (0, 0, 0)),
            pl.BlockSpec((DEPTH, DIM), rep2),
            pl.BlockSpec((DEPTH, DIM, DIM), lambda i: (0, 0, 0)),
            pl.BlockSpec((DEPTH, DIM), rep2),
            pl.BlockSpec((DIM,), rep1),
            pl.BlockSpec((DIM,), rep1),
        ],
        out_specs=pl.BlockSpec((GB, SP, DIM), lambda i: (i, 0, 0)),
        out_shape=jax.ShapeDtypeStruct((batch, SP, DIM), jnp.float32),
    )(g, c, pe, cmask, evo_w, evo_b, dec_w, dec_b, ln_g, ln_b)


def kernel(input_ids, emb_table, q_amps, ent_mat, evo_W, evo_b, dec_W, dec_b, ln_g, ln_b):
    batch, s = input_ids.shape
    assert s == S and batch % NW == 0

    ids_pad = jnp.concatenate(
        [input_ids.astype(jnp.int32),
         jnp.zeros((batch, SP - s), dtype=jnp.int32)], axis=1)
    tab = jnp.concatenate(
        [emb_table, q_amps[:, :, 0], q_amps[:, :, 1]], axis=1)
    ent_pad = jnp.pad(ent_mat, ((0, 0), (0, VP - VOCAB)))

    g, c = _sc_gather(ids_pad, tab, ent_pad, batch)
    c = c.reshape(batch, SP, SP)

    pe, cmask = _np_consts(s)
    out = _tc_dense(s, batch, g, c, pe, cmask, evo_W, evo_b, dec_W, dec_b,
                    ln_g, ln_b)
    return out[:, :s, :]


# speedup vs baseline: 1.8281x; 1.2650x over previous
import functools

import numpy as np
import jax
import jax.numpy as jnp
from jax import lax
from jax.experimental import pallas as pl
from jax.experimental.pallas import tpu as pltpu
from jax.experimental.pallas import tpu_sc as plsc

VOCAB = 1000
DIM = 128
DEPTH = 3
MAXPOS = 512
DECO = 0.1
S = 50
SP = 56
IDP = 64
VP = 1024
NC, NS, LANES = 2, 16, 16
NW = NC * NS
GB = 8
TW = 256
EW = VP // 2


def _pos_encoding_np(max_len, d):
    position = np.arange(max_len, dtype=np.float32)[:, None]
    div_term = np.exp(np.arange(0, d, 2, dtype=np.float32) * -(np.log(10000.0) / d))
    pe = np.zeros((max_len, d), dtype=np.float32)
    pe[:, 0::2] = np.sin(position * div_term)
    pe[:, 1::2] = np.cos(position * div_term)
    quantum_phase = np.sin(position * np.pi / max_len)
    pe = pe * (1.0 + 0.1 * quantum_phase)
    return pe


def _np_consts(s):
    pe = np.zeros((SP, DIM), dtype=np.float32)
    pe[:s] = _pos_encoding_np(MAXPOS, DIM)[:s]
    pos = np.arange(s, dtype=np.float32)
    dist = np.abs(pos[None, :] - pos[:, None])
    decay = np.exp(-DECO * dist) * (1.0 - np.eye(s, dtype=np.float32))
    cmask = np.zeros((SP, SP), dtype=np.float32)
    cmask[:s, :s] = decay
    return jnp.asarray(pe), jnp.asarray(cmask)


def _pack_bf16(x):
    n, m = x.shape
    b = x.astype(jnp.bfloat16).reshape(n, m // 2, 2)
    return lax.bitcast_convert_type(b, jnp.int32)


def _sc_body(bpw, ids_hbm, tab_hbm, ent_hbm, g_hbm, c_hbm,
             idv, tbuf0, tbuf1, ebuf0, ebuf1, cbuf0, cbuf1,
             semt0, semt1, seme0, seme1, semg0, semg1, semc0, semc1):
    wid = lax.axis_index("s") * NC + lax.axis_index("c")
    base = wid * bpw
    tbufs = (tbuf0, tbuf1)
    cbufs = (cbuf0, cbuf1)
    semt = (semt0, semt1)
    semg = (semg0, semg1)
    semc = (semc0, semc1)
    zeros = jnp.zeros((LANES,), jnp.float32)

    def idx(k, off, n):
        return idv.at[pl.ds(k * IDP + off, n)]

    def issue_t(k, slot):
        return pltpu.async_copy(tab_hbm.at[idx(k, 0, SP)], tbufs[slot], semt[slot])

    def issue_e0(k):
        return pltpu.async_copy(ent_hbm.at[idx(k, 0, 32)], ebuf0, seme0)

    def issue_e1(k):
        return pltpu.async_copy(ent_hbm.at[idx(k, 32, 24)], ebuf1, seme1)

    pltpu.sync_copy(ids_hbm.at[pl.ds(base * IDP, bpw * IDP)], idv)

    def ztail(j, c):
        cbuf0[pl.ds(S * SP + j * LANES, LANES)] = zeros
        cbuf1[pl.ds(S * SP + j * LANES, LANES)] = zeros
        return c
    lax.fori_loop(0, (SP - S) * SP // LANES, ztail, 0)

    issue_t(0, 0)
    issue_e0(0)
    issue_e1(0)

    def compress(ebuf, nrows, row0, k, cb):
        offs = (0, 16, 32, 40)
        cols = [idv[pl.ds(k * IDP + o, LANES)] for o in offs]
        chs = [c >> 1 for c in cols]
        sels = [(c & 1) == 1 for c in cols]

        def row(i, c2):
            spl = jnp.full((LANES,), i, dtype=jnp.int32)
            for g in range(4):
                v = plsc.load_gather(ebuf, [spl, chs[g]])
                lo = v << 16
                hi = v & jnp.int32(-65536)
                f = plsc.bitcast(jnp.where(sels[g], hi, lo), jnp.float32)
                cb[pl.ds((row0 + i) * SP + offs[g], LANES)] = f
            return c2
        lax.fori_loop(0, nrows, row, 0)

    def step(k, par):
        b = base + k
        kn = jnp.minimum(k + 1, bpw - 1)

        @pl.when(k >= 2)
        def _():
            pltpu.make_async_copy(cbufs[par], c_hbm.at[b - 2], semc[par]).wait()

        pltpu.make_async_copy(ent_hbm.at[idx(k, 0, 32)], ebuf0, seme0).wait()
        compress(ebuf0, 32, 0, k, cbufs[par])
        issue_e0(kn)
        pltpu.make_async_copy(ent_hbm.at[idx(k, 32, 24)], ebuf1, seme1).wait()
        compress(ebuf1, S - 32, 32, k, cbufs[par])
        issue_e1(kn)
        pltpu.async_copy(cbufs[par], c_hbm.at[b], semc[par])

        pltpu.make_async_copy(tab_hbm.at[idx(k, 0, SP)], tbufs[par], semt[par]).wait()

        @pl.when(k >= 1)
        def _():
            pltpu.make_async_copy(tbufs[1 - par], g_hbm.at[b - 1], semg[1 - par]).wait()

        issue_t(kn, 1 - par)
        pltpu.async_copy(tbufs[par], g_hbm.at[b], semg[par])

    def pair(kk, c):
        step(kk * 2, 0)
        step(kk * 2 + 1, 1)
        return c
    lax.fori_loop(0, bpw // 2, pair, 0)

    last = bpw - 1
    pltpu.make_async_copy(tab_hbm.at[idx(last, 0, SP)], tbufs[0], semt[0]).wait()
    pltpu.make_async_copy(ent_hbm.at[idx(last, 0, 32)], ebuf0, seme0).wait()
    pltpu.make_async_copy(ent_hbm.at[idx(last, 32, 24)], ebuf1, seme1).wait()
    pltpu.make_async_copy(tbufs[1], g_hbm.at[base + last], semg[1]).wait()
    pltpu.make_async_copy(cbufs[0], c_hbm.at[base + last - 1], semc[0]).wait()
    pltpu.make_async_copy(cbufs[1], c_hbm.at[base + last], semc[1]).wait()


def _sc_gather(ids_flat, tabp, entp, batch):
    bpw = batch // NW
    mesh = plsc.VectorSubcoreMesh(core_axis_name="c", subcore_axis_name="s")
    fn = pl.kernel(
        functools.partial(_sc_body, bpw),
        out_type=[
            jax.ShapeDtypeStruct((batch, SP, TW), jnp.int32),
            jax.ShapeDtypeStruct((batch, SP * SP), jnp.float32),
        ],
        mesh=mesh,
        compiler_params=pltpu.CompilerParams(needs_layout_passes=False),
        scratch_types=[
            pltpu.VMEM((batch // NW * IDP,), jnp.int32),
            pltpu.VMEM((SP, TW), jnp.int32),
            pltpu.VMEM((SP, TW), jnp.int32),
            pltpu.VMEM((32, EW), jnp.int32),
            pltpu.VMEM((24, EW), jnp.int32),
            pltpu.VMEM((SP * SP,), jnp.float32),
            pltpu.VMEM((SP * SP,), jnp.float32),
        ] + [pltpu.SemaphoreType.DMA] * 8,
    )
    return fn(ids_flat, tabp, entp)


def _tc_body(s, g_ref, c_ref, pe_ref, cmask_ref, evo_w_ref, evo_b_ref,
             dec_w_ref, dec_b_ref, ln_g_ref, ln_b_ref, out_ref):
    g = g_ref[...]
    emb = g[:, :, :DIM].astype(jnp.float32)
    qr = g[:, :, DIM:2 * DIM].astype(jnp.float32)
    qi = g[:, :, 2 * DIM:3 * DIM].astype(jnp.float32)
    pe = pe_ref[...]
    rowm = (lax.broadcasted_iota(jnp.int32, (SP, 1), 0) < s).astype(jnp.float32)

    x0 = emb + pe[None]
    real = x0 + qr
    mag = jnp.sqrt(real * real + qi * qi + 1e-8)
    x = x0 + 0.1 * mag * jnp.tanh(real)
    x = x * rowm[None]

    cm = c_ref[...] * cmask_ref[...][None]
    dn = (((1,), (0,)), ((), ()))
    ys = [lax.dot_general(cm[k], x[k], dn, preferred_element_type=jnp.float32)
          for k in range(GB)]
    x = x + 0.1 * jnp.stack(ys)

    xf = x.reshape(GB * SP, DIM)
    dnt = (((1,), (1,)), ((), ()))
    for step in range(DEPTH):
        ev = (lax.dot_general(xf, evo_w_ref[step], dnt,
                              preferred_element_type=jnp.float32)
              + evo_b_ref[step][None])
        gate = jax.nn.sigmoid(
            lax.dot_general(xf, dec_w_ref[step], dnt,
                            preferred_element_type=jnp.float32)
            + dec_b_ref[step][None])
        xf = ev * gate + xf * (1.0 - gate)

    mu = jnp.mean(xf, axis=-1, keepdims=True)
    var = jnp.mean((xf - mu) ** 2, axis=-1, keepdims=True)
    out = (xf - mu) * lax.rsqrt(var + 1e-5) * ln_g_ref[...][None] + ln_b_ref[...][None]
    out_ref[...] = out.reshape(GB, SP, DIM)


def _tc_dense(s, batch, g, c, pe, cmask, evo_w, evo_b, dec_w, dec_b, ln_g, ln_b):
    grid = (batch // GB,)
    rep2 = lambda i: (0, 0)
    rep1 = lambda i: (0,)
    return pl.pallas_call(
        functools.partial(_tc_body, s),
        grid=grid,
        in_specs=[
            pl.BlockSpec((GB, SP, 2 * TW), lambda i: (i, 0, 0)),
            pl.BlockSpec((GB, SP, SP), lambda i: (i, 0, 0)),
            pl.BlockSpec((SP, DIM), rep2),
            pl.BlockSpec((SP, SP), rep2),
            pl.BlockSpec((DEPTH, DIM, DIM), lambda i: (0, 0, 0)),
            pl.BlockSpec((DEPTH, DIM), rep2),
            pl.BlockSpec((DEPTH, DIM, DIM), lambda i: (0, 0, 0)),
            pl.BlockSpec((DEPTH, DIM), rep2),
            pl.BlockSpec((DIM,), rep1),
            pl.BlockSpec((DIM,), rep1),
        ],
        out_specs=pl.BlockSpec((GB, SP, DIM), lambda i: (i, 0, 0)),
        out_shape=jax.ShapeDtypeStruct((batch, SP, DIM), jnp.float32),
    )(g, c, pe, cmask, evo_w, evo_b, dec_w, dec_b, ln_g, ln_b)


def kernel(input_ids, emb_table, q_amps, ent_mat, evo_W, evo_b, dec_W, dec_b, ln_g, ln_b):
    batch, s = input_ids.shape
    assert s == S and batch % NW == 0

    ids_pad = jnp.concatenate(
        [input_ids.astype(jnp.int32),
         jnp.zeros((batch, IDP - s), dtype=jnp.int32)], axis=1).reshape(-1)
    tabp = _pack_bf16(jnp.concatenate(
        [emb_table, q_amps[:, :, 0], q_amps[:, :, 1],
         jnp.zeros((VOCAB, 2 * TW - 3 * DIM), jnp.float32)], axis=1))
    entp = _pack_bf16(jnp.pad(ent_mat, ((0, 0), (0, VP - VOCAB))))

    gp, c = _sc_gather(ids_pad, tabp, entp, batch)
    g = lax.bitcast_convert_type(gp, jnp.bfloat16).reshape(batch, SP, 2 * TW)
    c = c.reshape(batch, SP, SP)

    pe, cmask = _np_consts(s)
    out = _tc_dense(s, batch, g, c, pe, cmask, evo_W, evo_b, dec_W, dec_b,
                    ln_g, ln_b)
    return out[:, :s, :]


# speedup vs baseline: 5.2068x; 2.8482x over previous
import functools

import numpy as np
import jax
import jax.numpy as jnp
from jax import lax
from jax.experimental import pallas as pl
from jax.experimental.pallas import tpu as pltpu
from jax.experimental.pallas import tpu_sc as plsc

VOCAB = 1000
DIM = 128
DEPTH = 3
MAXPOS = 512
DECO = 0.1
S = 50
SP = 56
IDP = 64
VP = 1024
NC, NS, LANES = 2, 16, 16
NW = NC * NS
GB = 8
TW = 256
EW = VP // 2
E0R = 32
E1R = S - E0R


def _pos_encoding_np(max_len, d):
    position = np.arange(max_len, dtype=np.float32)[:, None]
    div_term = np.exp(np.arange(0, d, 2, dtype=np.float32) * -(np.log(10000.0) / d))
    pe = np.zeros((max_len, d), dtype=np.float32)
    pe[:, 0::2] = np.sin(position * div_term)
    pe[:, 1::2] = np.cos(position * div_term)
    quantum_phase = np.sin(position * np.pi / max_len)
    pe = pe * (1.0 + 0.1 * quantum_phase)
    return pe


def _np_consts(s):
    pe = jnp.asarray(_pos_encoding_np(MAXPOS, DIM)[:s])
    pos = np.arange(s, dtype=np.float32)
    dist = np.abs(pos[None, :] - pos[:, None])
    decay = np.exp(-DECO * dist) * (1.0 - np.eye(s, dtype=np.float32))
    cmask = np.zeros((s, SP), dtype=np.float32)
    cmask[:, :s] = decay
    return pe, jnp.asarray(cmask)


def _sc_body(bpw, ids_hbm, tab_hbm, ent_hbm, g_hbm, c_hbm,
             idv, tbuf0, tbuf1, e0a, e0b, e1a, e1b, cbuf0, cbuf1,
             semt0, semt1, seme0a, seme0b, seme1a, seme1b,
             semg0, semg1, semc0, semc1):
    wid = lax.axis_index("s") * NC + lax.axis_index("c")
    base = wid * bpw
    tbufs = (tbuf0, tbuf1)
    e0s = (e0a, e0b)
    e1s = (e1a, e1b)
    cbufs = (cbuf0, cbuf1)
    semt = (semt0, semt1)
    seme0 = (seme0a, seme0b)
    seme1 = (seme1a, seme1b)
    semg = (semg0, semg1)
    semc = (semc0, semc1)

    def idx(k, off, n):
        return idv.at[pl.ds(k * IDP + off, n)]

    def issue_t(k, slot):
        pltpu.async_copy(tab_hbm.at[idx(k, 0, S)], tbufs[slot], semt[slot])

    def issue_e(k, slot):
        pltpu.async_copy(ent_hbm.at[idx(k, 0, E0R)], e0s[slot], seme0[slot])
        pltpu.async_copy(ent_hbm.at[idx(k, E0R, E1R)], e1s[slot], seme1[slot])

    pltpu.sync_copy(ids_hbm.at[pl.ds(base * IDP, bpw * IDP)], idv)

    issue_t(0, 0)
    issue_e(0, 0)

    def compress(ebuf, nrows, row0, k, cb):
        offs = (0, 16, 32, 40)
        cols = [idv[pl.ds(k * IDP + o, LANES)] for o in offs]
        chs = [c >> 1 for c in cols]
        sels = [(c & 1) == 1 for c in cols]

        def row(i, c2):
            spl = jnp.full((LANES,), i, dtype=jnp.int32)
            for g in range(4):
                v = plsc.load_gather(ebuf, [spl, chs[g]])
                lo = v << 16
                hi = v & jnp.int32(-65536)
                f = plsc.bitcast(jnp.where(sels[g], hi, lo), jnp.float32)
                cb[pl.ds((row0 + i) * SP + offs[g], LANES)] = f
            return c2
        lax.fori_loop(0, nrows, row, 0)

    def step(k, par):
        b = base + k
        kn = jnp.minimum(k + 1, bpw - 1)

        @pl.when(k >= 1)
        def _():
            pltpu.make_async_copy(tbufs[1 - par], g_hbm.at[b - 1], semg[1 - par]).wait()
        issue_t(kn, 1 - par)
        issue_e(kn, 1 - par)

        @pl.when(k >= 2)
        def _():
            pltpu.make_async_copy(cbufs[par], c_hbm.at[b - 2], semc[par]).wait()

        pltpu.make_async_copy(ent_hbm.at[idx(k, 0, E0R)], e0s[par], seme0[par]).wait()
        compress(e0s[par], E0R, 0, k, cbufs[par])
        pltpu.make_async_copy(ent_hbm.at[idx(k, E0R, E1R)], e1s[par], seme1[par]).wait()
        compress(e1s[par], E1R, E0R, k, cbufs[par])
        pltpu.async_copy(cbufs[par], c_hbm.at[b], semc[par])

        pltpu.make_async_copy(tab_hbm.at[idx(k, 0, S)], tbufs[par], semt[par]).wait()
        pltpu.async_copy(tbufs[par], g_hbm.at[b], semg[par])

    def pair(kk, c):
        step(kk * 2, 0)
        step(kk * 2 + 1, 1)
        return c
    lax.fori_loop(0, bpw // 2, pair, 0)

    last = bpw - 1
    pltpu.make_async_copy(tab_hbm.at[idx(last, 0, S)], tbufs[0], semt[0]).wait()
    pltpu.make_async_copy(ent_hbm.at[idx(last, 0, E0R)], e0s[0], seme0[0]).wait()
    pltpu.make_async_copy(ent_hbm.at[idx(last, E0R, E1R)], e1s[0], seme1[0]).wait()
    pltpu.make_async_copy(tbufs[1], g_hbm.at[base + last], semg[1]).wait()
    pltpu.make_async_copy(cbufs[0], c_hbm.at[base + last - 1], semc[0]).wait()
    pltpu.make_async_copy(cbufs[1], c_hbm.at[base + last], semc[1]).wait()


def _sc_gather(ids_flat, tabp, entp, batch):
    bpw = batch // NW
    mesh = plsc.VectorSubcoreMesh(core_axis_name="c", subcore_axis_name="s")
    fn = pl.kernel(
        functools.partial(_sc_body, bpw),
        out_type=[
            jax.ShapeDtypeStruct((batch, S, TW), jnp.int32),
            jax.ShapeDtypeStruct((batch, S * SP), jnp.float32),
        ],
        mesh=mesh,
        compiler_params=pltpu.CompilerParams(needs_layout_passes=False),
        scratch_types=[
            pltpu.VMEM((bpw * IDP,), jnp.int32),
            pltpu.VMEM((S, TW), jnp.int32),
            pltpu.VMEM((S, TW), jnp.int32),
            pltpu.VMEM((E0R, EW), jnp.int32),
            pltpu.VMEM((E0R, EW), jnp.int32),
            pltpu.VMEM((E1R, EW), jnp.int32),
            pltpu.VMEM((E1R, EW), jnp.int32),
            pltpu.VMEM((S * SP,), jnp.float32),
            pltpu.VMEM((S * SP,), jnp.float32),
        ] + [pltpu.SemaphoreType.DMA] * 10,
    )
    return fn(ids_flat, tabp, entp)


def _tc_body(s, g_ref, c_ref, pe_ref, cmask_ref, evo_w_ref, evo_b_ref,
             dec_w_ref, dec_b_ref, ln_g_ref, ln_b_ref, out_ref):
    gw = g_ref[...]
    va = lax.bitcast_convert_type(gw << 16, jnp.float32)
    vb = lax.bitcast_convert_type(gw & jnp.int32(-65536), jnp.float32)
    emb = va[:, :, :DIM]
    qr = va[:, :, DIM:]
    qi = vb[:, :, :DIM]
    pe = pe_ref[...]

    x0 = emb + pe[None]
    real = x0 + qr
    mag = jnp.sqrt(real * real + qi * qi + 1e-8)
    x = x0 + 0.1 * mag * jnp.tanh(real)

    cm = c_ref[...] * cmask_ref[...][None]
    xpad = jnp.concatenate(
        [x, jnp.zeros((GB, SP - s, DIM), jnp.float32)], axis=1)
    dn = (((1,), (0,)), ((), ()))
    ys = [lax.dot_general(cm[k], xpad[k], dn, preferred_element_type=jnp.float32)
          for k in range(GB)]
    x = x + 0.1 * jnp.stack(ys)

    xf = x.reshape(GB * s, DIM)
    dnt = (((1,), (1,)), ((), ()))
    for step in range(DEPTH):
        ev = (lax.dot_general(xf, evo_w_ref[step], dnt,
                              preferred_element_type=jnp.float32)
              + evo_b_ref[step][None])
        gate = jax.nn.sigmoid(
            lax.dot_general(xf, dec_w_ref[step], dnt,
                            preferred_element_type=jnp.float32)
            + dec_b_ref[step][None])
        xf = ev * gate + xf * (1.0 - gate)

    mu = jnp.mean(xf, axis=-1, keepdims=True)
    var = jnp.mean((xf - mu) ** 2, axis=-1, keepdims=True)
    out = (xf - mu) * lax.rsqrt(var + 1e-5) * ln_g_ref[...][None] + ln_b_ref[...][None]
    out_ref[...] = out.reshape(GB, s, DIM)


def _tc_dense(s, batch, g, c, pe, cmask, evo_w, evo_b, dec_w, dec_b, ln_g, ln_b):
    grid = (batch // GB,)
    rep2 = lambda i: (0, 0)
    rep1 = lambda i: (0,)
    return pl.pallas_call(
        functools.partial(_tc_body, s),
        grid=grid,
        in_specs=[
            pl.BlockSpec((GB, S, TW), lambda i: (i, 0, 0)),
            pl.BlockSpec((GB, S, SP), lambda i: (i, 0, 0)),
            pl.BlockSpec((S, DIM), rep2),
            pl.BlockSpec((S, SP), rep2),
            pl.BlockSpec((DEPTH, DIM, DIM), lambda i: (0, 0, 0)),
            pl.BlockSpec((DEPTH, DIM), rep2),
            pl.BlockSpec((DEPTH, DIM, DIM), lambda i: (0, 0, 0)),
            pl.BlockSpec((DEPTH, DIM), rep2),
            pl.BlockSpec((DIM,), rep1),
            pl.BlockSpec((DIM,), rep1),
        ],
        out_specs=pl.BlockSpec((GB, S, DIM), lambda i: (i, 0, 0)),
        out_shape=jax.ShapeDtypeStruct((batch, S, DIM), jnp.float32),
    )(g, c, pe, cmask, evo_w, evo_b, dec_w, dec_b, ln_g, ln_b)


def kernel(input_ids, emb_table, q_amps, ent_mat, evo_W, evo_b, dec_W, dec_b, ln_g, ln_b):
    batch, s = input_ids.shape
    assert s == S and batch % NW == 0

    ids_pad = jnp.concatenate(
        [input_ids.astype(jnp.int32),
         jnp.zeros((batch, IDP - s), dtype=jnp.int32)], axis=1).reshape(-1)
    col_a = jnp.concatenate([emb_table, q_amps[:, :, 0]], axis=1).astype(jnp.bfloat16)
    col_b = jnp.concatenate(
        [q_amps[:, :, 1].astype(jnp.bfloat16),
         jnp.zeros((VOCAB, DIM), jnp.bfloat16)], axis=1)
    tabp = lax.bitcast_convert_type(
        jnp.stack([col_a, col_b], axis=-1), jnp.int32)
    entp = lax.bitcast_convert_type(
        jnp.pad(ent_mat.astype(jnp.bfloat16),
                ((0, 0), (0, VP - VOCAB))).reshape(VOCAB, EW, 2),
        jnp.int32)

    g, c = _sc_gather(ids_pad, tabp, entp, batch)
    c = c.reshape(batch, S, SP)

    pe, cmask = _np_consts(s)
    return _tc_dense(s, batch, g, c, pe, cmask, evo_W, evo_b, dec_W, dec_b,
                     ln_g, ln_b)
